# Initial kernel scaffold; baseline (speedup 1.0000x reference)
import jax
import jax.numpy as jnp
from jax.experimental import pallas as pl


def kernel(X, W_sp, W_sd, W_mp, W_md, user_ratings, user_personalities, top_map, mid_map):
    raise NotImplementedError("write your pallas kernel here")


# speedup vs baseline: 4.1754x; 4.1754x over previous
import functools

import jax
import jax.numpy as jnp
from jax import lax
from jax.experimental import pallas as pl
from jax.experimental.pallas import tpu as pltpu

B = 1024
D = 32
H = 64
N_ITEMS = 100000
N_TOP = 2000
N_MID = 10000
N_USERS = 256
K = 20

BB = 256
CHUNK = 2048
N_CHUNKS = (N_ITEMS + CHUNK - 1) // CHUNK
PAD_LANES = 128
IDX_SENT = 2**31 - 1
MAP_SENT = 1 << 29
NEG_INF = float("-inf")


def _topk_extract(V, I, k):
    outs_v, outs_i = [], []
    for _ in range(k):
        m = jnp.max(V, axis=1, keepdims=True)
        tie = V == m
        ci = jnp.where(tie, I, IDX_SENT)
        si = jnp.min(ci, axis=1, keepdims=True)
        outs_v.append(m)
        outs_i.append(si)
        V = jnp.where(tie & (I == si), NEG_INF, V)
    return jnp.concatenate(outs_v, axis=1), jnp.concatenate(outs_i, axis=1)


def _subset_kernel(x_ref, wp_ref, wd_ref, map_ref, zc_ref, out_ref):
    h = jnp.tanh(
        lax.dot_general(x_ref[...], wp_ref[...], (((1,), (0,)), ((), ())),
                        preferred_element_type=jnp.float32))
    preds = lax.dot_general(h, wd_ref[...], (((1,), (0,)), ((), ())),
                            preferred_element_type=jnp.float32)
    bb = preds.shape[0]
    V = jnp.concatenate([preds, jnp.zeros((bb, PAD_LANES), jnp.float32)], axis=1)
    I = jnp.concatenate([
        jnp.broadcast_to(map_ref[...], (bb, map_ref.shape[-1])),
        jnp.broadcast_to(zc_ref[...], (bb, PAD_LANES)),
    ], axis=1)
    _, idx = _topk_extract(V, I, K)
    out_ref[...] = jnp.concatenate(
        [idx, jnp.full((bb, PAD_LANES - K), IDX_SENT, jnp.int32)], axis=1)


def _subset_topk(X, W_p, W_d, idx_map, zc, n_sub_pad):
    call = pl.pallas_call(
        _subset_kernel,
        grid=(B // BB,),
        in_specs=[
            pl.BlockSpec((BB, D), lambda b: (b, 0)),
            pl.BlockSpec((D, H), lambda b: (0, 0)),
            pl.BlockSpec((H, n_sub_pad), lambda b: (0, 0)),
            pl.BlockSpec((1, n_sub_pad), lambda b: (0, 0)),
            pl.BlockSpec((1, PAD_LANES), lambda b: (0, 0)),
        ],
        out_specs=pl.BlockSpec((BB, PAD_LANES), lambda b: (b, 0)),
        out_shape=jax.ShapeDtypeStruct((B, PAD_LANES), jnp.int32),
    )
    return call(X, W_p, W_d, idx_map, zc)[:, :K]


def _knn_kernel(x_ref, p_ref, r_ref, out_ref, w_ref, vals_ref):
    b, j = pl.program_id(0), pl.program_id(1)

    @pl.when(j == 0)
    def _():
        x = x_ref[...]
        xn = x / (jnp.sqrt(jnp.sum(x * x, axis=1, keepdims=True)) + 1e-8)
        p = p_ref[...]
        pn = p / (jnp.sqrt(jnp.sum(p * p, axis=1, keepdims=True)) + 1e-8)
        sims = lax.dot_general(xn, pn, (((1,), (1,)), ((), ())),
                               preferred_element_type=jnp.float32)
        w_ref[...] = jnp.maximum(sims, 0.0)

    scores = lax.dot_general(w_ref[...], r_ref[...], (((1,), (0,)), ((), ())),
                             preferred_element_type=jnp.float32)
    gcol = j * CHUNK + lax.broadcasted_iota(jnp.int32, (BB, CHUNK), 1)
    scores = jnp.where(gcol < N_ITEMS, scores, NEG_INF)

    first = j == 0
    prev_v = jnp.where(first, NEG_INF, vals_ref[...])
    prev_i = jnp.where(first, IDX_SENT, out_ref[...])
    V = jnp.concatenate([prev_v, scores], axis=1)
    I = jnp.concatenate([prev_i, gcol], axis=1)
    vals, idx = _topk_extract(V, I, K)
    vals_ref[...] = jnp.concatenate(
        [vals, jnp.full((BB, PAD_LANES - K), NEG_INF, jnp.float32)], axis=1)
    out_ref[...] = jnp.concatenate(
        [idx, jnp.full((BB, PAD_LANES - K), IDX_SENT, jnp.int32)], axis=1)


def _knn_topk(X, user_ratings, user_personalities):
    call = pl.pallas_call(
        _knn_kernel,
        grid=(B // BB, N_CHUNKS),
        in_specs=[
            pl.BlockSpec((BB, D), lambda b, j: (b, 0)),
            pl.BlockSpec((N_USERS, D), lambda b, j: (0, 0)),
            pl.BlockSpec((N_USERS, CHUNK), lambda b, j: (0, j)),
        ],
        out_specs=pl.BlockSpec((BB, PAD_LANES), lambda b, j: (b, 0)),
        out_shape=jax.ShapeDtypeStruct((B, PAD_LANES), jnp.int32),
        scratch_shapes=[
            pltpu.VMEM((BB, N_USERS), jnp.float32),
            pltpu.VMEM((BB, PAD_LANES), jnp.float32),
        ],
    )
    return call(X, user_personalities, user_ratings)[:, :K]


def _smallest_unmapped(idx_map):
    present = jnp.zeros((N_ITEMS,), jnp.int32).at[idx_map].set(1)
    score = jnp.arange(N_ITEMS, dtype=jnp.int32) + present * (2 * N_ITEMS)
    neg_top, _ = lax.top_k(-score, PAD_LANES)
    return (-neg_top).reshape(1, PAD_LANES)


def kernel(X, W_sp, W_sd, W_mp, W_md, user_ratings, user_personalities,
           top_map, mid_map):
    top_map = top_map.astype(jnp.int32)
    mid_map = mid_map.astype(jnp.int32)

    n_top_pad = 2048
    n_mid_pad = 10112
    W_sd_p = jnp.pad(W_sd, ((0, 0), (0, n_top_pad - N_TOP)))
    W_md_p = jnp.pad(W_md, ((0, 0), (0, n_mid_pad - N_MID)))
    top_map_p = jnp.pad(top_map, (0, n_top_pad - N_TOP),
                        constant_values=MAP_SENT).reshape(1, n_top_pad)
    mid_map_p = jnp.pad(mid_map, (0, n_mid_pad - N_MID),
                        constant_values=MAP_SENT).reshape(1, n_mid_pad)
    zc_top = _smallest_unmapped(top_map)
    zc_mid = _smallest_unmapped(mid_map)

    top_idx = _subset_topk(X, W_sp, W_sd_p, top_map_p, zc_top, n_top_pad)
    mid_idx = _subset_topk(X, W_mp, W_md_p, mid_map_p, zc_mid, n_mid_pad)
    k_idx = _knn_topk(X, user_ratings, user_personalities)

    return jnp.concatenate(
        [top_idx[:, None, :], mid_idx[:, None, :], k_idx[:, None, :]], axis=1)
